# baseline (speedup 1.0000x reference)
import functools

import jax
import jax.numpy as jnp
from jax import lax
from jax.experimental import pallas as pl
from jax.experimental.pallas import tpu as pltpu
from jax.experimental.pallas import tpu_sc as plsc

_N = 16384
_K = 50
_D = 60
_DV = 40
_NC = 2
_NS = 16
_NW = _NC * _NS
_NB = _N // _NW
_NV = _NB // 16
_TSTRIDE = _D * 18 + 1


def _make_expand():
    mesh = plsc.VectorSubcoreMesh(core_axis_name="c", subcore_axis_name="s")

    @functools.partial(
        pl.kernel,
        mesh=mesh,
        compiler_params=pltpu.CompilerParams(needs_layout_passes=False),
        out_type=jax.ShapeDtypeStruct((_K, _D, _N), jnp.float32),
        scratch_types=[
            pltpu.VMEM((_K, _NB), jnp.int32),
            pltpu.VMEM((_N,), jnp.int32),
            pltpu.VMEM((16 * _TSTRIDE,), jnp.float32),
            pltpu.VMEM((_D, _NB), jnp.float32),
            pltpu.VMEM((_D, _NB), jnp.float32),
            pltpu.SemaphoreType.DMA,
        ],
    )
    def expand(tops_hbm, spk_hbm, tab_hbm, out_hbm, t_v, spk_v, tab_v, stg0, stg1, sem):
        wid = lax.axis_index("s") * _NC + lax.axis_index("c")
        nb = wid * _NB
        pltpu.sync_copy(spk_hbm, spk_v)
        pltpu.sync_copy(tab_hbm, tab_v)
        pltpu.sync_copy(tops_hbm.at[:, pl.ds(nb, _NB)], t_v)

        lane = lax.iota(jnp.int32, 16)
        lane_off = lane * _TSTRIDE
        stgs = (stg0, stg1)

        @plsc.parallel_loop(0, _NV, unroll=2)
        def const_body(j):
            for stg in stgs:
                for c in range(_DV, _D):
                    stg[c, pl.ds(j * 16, 16)] = plsc.load_gather(
                        tab_v, [lane_off + c * 18]
                    )

        def fill(k, stg):
            @plsc.parallel_loop(0, _NV, unroll=2)
            def jbody(j):
                n_vec = nb + j * 16 + lane
                t = t_v[k, pl.ds(j * 16, 16)]
                spk_t = plsc.load_gather(spk_v, [t])
                spk_n = spk_v[pl.ds(nb + j * 16, 16)]
                d = jnp.maximum(jnp.abs(n_vec - t), 1)
                e = (lax.bitcast_convert_type(d.astype(jnp.float32), jnp.int32) >> 23) - 127
                dist = jnp.where(d < 5, d - 1, jnp.minimum(e, 6) + 2)
                cidx = lane_off + jnp.where(spk_t == spk_n, dist + 9, dist)
                for c in range(_DV):
                    stg[c, pl.ds(j * 16, 16)] = plsc.load_gather(
                        tab_v, [cidx + c * 18]
                    )

        def k2body(k2, carry):
            for b in range(2):
                k = k2 * 2 + b
                stg = stgs[b]

                @pl.when(k >= 2)
                def _wait():
                    pltpu.make_async_copy(
                        stg, out_hbm.at[b, :, pl.ds(nb, _NB)], sem
                    ).wait()

                fill(k, stg)
                pltpu.async_copy(stg, out_hbm.at[k, :, pl.ds(nb, _NB)], sem)
            return carry

        lax.fori_loop(0, _K // 2, k2body, 0)

        for b in range(2):
            pltpu.make_async_copy(
                stgs[b], out_hbm.at[b, :, pl.ds(nb, _NB)], sem
            ).wait()

    return expand


_EXPAND = _make_expand()


def kernel(top_indices, speaker_map, genre, word_ids, genre_emb, distance_emb, speaker_emb):
    del word_ids
    spk_part = jnp.repeat(speaker_emb, 9, axis=0)
    dist_part = jnp.tile(distance_emb, (2, 1))
    genre_part = jnp.broadcast_to(genre_emb[genre][None, :], (18, 20))
    table = jnp.concatenate([spk_part, dist_part, genre_part], axis=1)
    tab_one = jnp.pad(table.T.reshape(_D * 18), (0, 1))
    tab_flat = jnp.tile(tab_one, 16)
    tops_t = top_indices.astype(jnp.int32).T
    spk = speaker_map.astype(jnp.int32)
    out = _EXPAND(tops_t, spk, tab_flat)
    return out.transpose(2, 0, 1)


# speedup vs baseline: 1.4792x; 1.4792x over previous
import functools

import jax
import jax.numpy as jnp
from jax import lax
from jax.experimental import pallas as pl
from jax.experimental.pallas import tpu as pltpu
from jax.experimental.pallas import tpu_sc as plsc

_N = 16384
_K = 50
_D = 60
_DV = 40
_NC = 2
_NS = 16
_NW = _NC * _NS
_NB = _N // _NW
_NV = _NB // 16


def _make_expand():
    mesh = plsc.VectorSubcoreMesh(core_axis_name="c", subcore_axis_name="s")

    @functools.partial(
        pl.kernel,
        mesh=mesh,
        compiler_params=pltpu.CompilerParams(needs_layout_passes=False),
        out_type=jax.ShapeDtypeStruct((_K, _D, _N), jnp.float32),
        scratch_types=[
            pltpu.VMEM((_K, _NB), jnp.int32),
            pltpu.VMEM((_N,), jnp.int32),
            pltpu.VMEM((_D * 18,), jnp.float32),
            pltpu.VMEM((_D, _NB), jnp.float32),
            pltpu.VMEM((_D, _NB), jnp.float32),
            pltpu.SemaphoreType.DMA,
        ],
    )
    def expand(tops_hbm, spk_hbm, tab_hbm, out_hbm, t_v, spk_v, tab_v, stg0, stg1, sem):
        wid = lax.axis_index("s") * _NC + lax.axis_index("c")
        nb = wid * _NB
        pltpu.sync_copy(spk_hbm, spk_v)
        pltpu.sync_copy(tab_hbm, tab_v)
        pltpu.sync_copy(tops_hbm.at[:, pl.ds(nb, _NB)], t_v)

        lane = lax.iota(jnp.int32, 16)
        stgs = (stg0, stg1)

        @plsc.parallel_loop(0, _NV, unroll=2)
        def const_body(j):
            for stg in stgs:
                for c in range(_DV, _D):
                    stg[c, pl.ds(j * 16, 16)] = plsc.load_gather(
                        tab_v, [lane * 0 + c * 18]
                    )

        def fill(k, stg):
            @plsc.parallel_loop(0, _NV, unroll=2)
            def jbody(j):
                n_vec = nb + j * 16 + lane
                t = t_v[k, pl.ds(j * 16, 16)]
                spk_t = plsc.load_gather(spk_v, [t])
                spk_n = spk_v[pl.ds(nb + j * 16, 16)]
                d = jnp.maximum(jnp.abs(n_vec - t), 1)
                e = (lax.bitcast_convert_type(d.astype(jnp.float32), jnp.int32) >> 23) - 127
                dist = jnp.where(d < 5, d - 1, jnp.minimum(e, 6) + 2)
                cidx = jnp.where(spk_t == spk_n, dist + 9, dist)
                for c in range(_DV):
                    stg[c, pl.ds(j * 16, 16)] = plsc.load_gather(
                        tab_v, [cidx + c * 18]
                    )

        def k2body(k2, carry):
            for b in range(2):
                k = k2 * 2 + b
                stg = stgs[b]

                @pl.when(k >= 2)
                def _wait():
                    pltpu.make_async_copy(
                        stg, out_hbm.at[b, :, pl.ds(nb, _NB)], sem
                    ).wait()

                fill(k, stg)
                pltpu.async_copy(stg, out_hbm.at[k, :, pl.ds(nb, _NB)], sem)
            return carry

        lax.fori_loop(0, _K // 2, k2body, 0)

        for b in range(2):
            pltpu.make_async_copy(
                stgs[b], out_hbm.at[b, :, pl.ds(nb, _NB)], sem
            ).wait()

    return expand


_EXPAND = _make_expand()


def kernel(top_indices, speaker_map, genre, word_ids, genre_emb, distance_emb, speaker_emb):
    del word_ids
    spk_part = jnp.repeat(speaker_emb, 9, axis=0)
    dist_part = jnp.tile(distance_emb, (2, 1))
    genre_part = jnp.broadcast_to(genre_emb[genre][None, :], (18, 20))
    table = jnp.concatenate([spk_part, dist_part, genre_part], axis=1)
    tab_flat = table.T.reshape(_D * 18)
    tops_t = top_indices.astype(jnp.int32).T
    spk = speaker_map.astype(jnp.int32)
    out = _EXPAND(tops_t, spk, tab_flat)
    return out.transpose(2, 0, 1)


# speedup vs baseline: 2.1910x; 1.4812x over previous
import functools

import jax
import jax.numpy as jnp
from jax import lax
from jax.experimental import pallas as pl
from jax.experimental.pallas import tpu as pltpu
from jax.experimental.pallas import tpu_sc as plsc

_N = 16384
_K = 50
_D = 60
_DV = 40
_NC = 2
_NS = 16
_NW = _NC * _NS
_NB = _N // _NW
_NV = _NB // 16


def _make_expand():
    mesh = plsc.VectorSubcoreMesh(core_axis_name="c", subcore_axis_name="s")

    @functools.partial(
        pl.kernel,
        mesh=mesh,
        compiler_params=pltpu.CompilerParams(needs_layout_passes=False),
        out_type=jax.ShapeDtypeStruct((_K, _D, _N), jnp.float32),
        scratch_types=[
            pltpu.VMEM((_K, _NB), jnp.int32),
            pltpu.VMEM((_N,), jnp.int32),
            pltpu.VMEM((_D * 18,), jnp.float32),
            pltpu.VMEM((_D, _NB), jnp.float32),
            pltpu.VMEM((_D, _NB), jnp.float32),
            pltpu.VMEM((_NB,), jnp.int32),
            pltpu.SemaphoreType.DMA,
        ],
    )
    def expand(tops_hbm, spk_hbm, tab_hbm, out_hbm, t_v, spk_v, tab_v, stg0, stg1, idx_v, sem):
        wid = lax.axis_index("s") * _NC + lax.axis_index("c")
        nb = wid * _NB
        pltpu.sync_copy(spk_hbm, spk_v)
        pltpu.sync_copy(tab_hbm, tab_v)
        pltpu.sync_copy(tops_hbm.at[:, pl.ds(nb, _NB)], t_v)

        lane = lax.iota(jnp.int32, 16)
        stgs = (stg0, stg1)

        @plsc.parallel_loop(0, _NV, unroll=2)
        def const_body(j):
            for stg in stgs:
                for c in range(_DV, _D):
                    stg[c, pl.ds(j * 16, 16)] = plsc.load_gather(
                        tab_v, [lane * 0 + c * 18]
                    )

        def fill(k, stg):
            @plsc.parallel_loop(0, _NV, unroll=2)
            def ibody(j):
                n_vec = nb + j * 16 + lane
                t = t_v[k, pl.ds(j * 16, 16)]
                spk_t = plsc.load_gather(spk_v, [t])
                spk_n = spk_v[pl.ds(nb + j * 16, 16)]
                d = jnp.maximum(jnp.abs(n_vec - t), 1)
                e = (lax.bitcast_convert_type(d.astype(jnp.float32), jnp.int32) >> 23) - 127
                dist = jnp.where(d < 5, d - 1, jnp.minimum(e, 6) + 2)
                idx_v[pl.ds(j * 16, 16)] = jnp.where(spk_t == spk_n, dist + 9, dist)

            @plsc.parallel_loop(0, _NV, unroll=2)
            def jbody(j):
                cidx = idx_v[pl.ds(j * 16, 16)]
                for c in range(_DV):
                    stg[c, pl.ds(j * 16, 16)] = plsc.load_gather(
                        tab_v, [cidx + c * 18]
                    )

        def k2body(k2, carry):
            for b in range(2):
                k = k2 * 2 + b
                stg = stgs[b]

                @pl.when(k >= 2)
                def _wait():
                    pltpu.make_async_copy(
                        stg, out_hbm.at[b, :, pl.ds(nb, _NB)], sem
                    ).wait()

                fill(k, stg)
                pltpu.async_copy(stg, out_hbm.at[k, :, pl.ds(nb, _NB)], sem)
            return carry

        lax.fori_loop(0, _K // 2, k2body, 0)

        for b in range(2):
            pltpu.make_async_copy(
                stgs[b], out_hbm.at[b, :, pl.ds(nb, _NB)], sem
            ).wait()

    return expand


_EXPAND = _make_expand()


def kernel(top_indices, speaker_map, genre, word_ids, genre_emb, distance_emb, speaker_emb):
    del word_ids
    spk_part = jnp.repeat(speaker_emb, 9, axis=0)
    dist_part = jnp.tile(distance_emb, (2, 1))
    genre_part = jnp.broadcast_to(genre_emb[genre][None, :], (18, 20))
    table = jnp.concatenate([spk_part, dist_part, genre_part], axis=1)
    tab_flat = table.T.reshape(_D * 18)
    tops_t = top_indices.astype(jnp.int32).T
    spk = speaker_map.astype(jnp.int32)
    out = _EXPAND(tops_t, spk, tab_flat)
    return out.transpose(2, 0, 1)
